# baseline (speedup 1.0000x reference)
import functools

import jax
import jax.numpy as jnp
from jax import lax
from jax.experimental import pallas as pl
from jax.experimental.pallas import tpu as pltpu
from jax.experimental.pallas import tpu_sc as plsc

_VOCAB = 1000000
_D = 64
_B = 4096 * 26
_NC, _NS = 2, 16
_NW = _NC * _NS
_B_PER_W = _B // _NW
_CHUNK = 128
_N_CHUNKS = _B_PER_W // _CHUNK

_mesh = plsc.VectorSubcoreMesh(core_axis_name="c", subcore_axis_name="s")


@functools.partial(
    pl.kernel,
    out_type=jax.ShapeDtypeStruct((_B, _D), jnp.float32),
    mesh=_mesh,
    scratch_types=[
        pltpu.VMEM((_N_CHUNKS, _CHUNK), jnp.int32),
        pltpu.VMEM((_CHUNK, _D), jnp.float32),
        pltpu.SemaphoreType.DMA,
    ],
    compiler_params=pltpu.CompilerParams(use_tc_tiling_on_sc=False),
)
def _gather_rows(idx_hbm, table_hbm, out_hbm, idx_v, rows_v, gsem):
    wid = lax.axis_index("s") * _NC + lax.axis_index("c")
    base = wid * _B_PER_W
    pltpu.sync_copy(idx_hbm.at[wid], idx_v)

    @pl.loop(0, _N_CHUNKS)
    def _chunk(c):
        pltpu.async_copy(table_hbm.at[idx_v.at[c]], rows_v, gsem).wait()
        pltpu.sync_copy(rows_v, out_hbm.at[pl.ds(base + c * _CHUNK, _CHUNK)])


def kernel(x, word_mat):
    idx = x.reshape(_NW, _N_CHUNKS, _CHUNK)
    out = _gather_rows(idx, word_mat)
    return out.reshape(x.shape[0], x.shape[1], _D)


# speedup vs baseline: 1.0282x; 1.0282x over previous
import functools

import jax
import jax.numpy as jnp
from jax import lax
from jax.experimental import pallas as pl
from jax.experimental.pallas import tpu as pltpu
from jax.experimental.pallas import tpu_sc as plsc

_VOCAB = 1000000
_D = 64
_B = 4096 * 26
_NC, _NS = 2, 16
_NW = _NC * _NS
_B_PER_W = _B // _NW
_CHUNK = 128
_N_CHUNKS = _B_PER_W // _CHUNK
_NBUF = 4
_LOOKAHEAD = _NBUF - 1

_mesh = plsc.VectorSubcoreMesh(core_axis_name="c", subcore_axis_name="s")


@functools.partial(
    pl.kernel,
    out_type=jax.ShapeDtypeStruct((_B, _D), jnp.float32),
    mesh=_mesh,
    scratch_types=[
        pltpu.VMEM((_N_CHUNKS, _CHUNK), jnp.int32),
        pltpu.VMEM((_NBUF, _CHUNK, _D), jnp.float32),
        pltpu.SemaphoreType.DMA((_NBUF,)),
        pltpu.SemaphoreType.DMA((_NBUF,)),
    ],
    compiler_params=pltpu.CompilerParams(use_tc_tiling_on_sc=False),
)
def _gather_rows(idx_hbm, table_hbm, out_hbm, idx_v, rows_v, gsem, osem):
    wid = lax.axis_index("s") * _NC + lax.axis_index("c")
    base = wid * _B_PER_W
    pltpu.sync_copy(idx_hbm.at[wid], idx_v)

    def gather(c):
        b = c % _NBUF
        return pltpu.make_async_copy(
            table_hbm.at[idx_v.at[c]], rows_v.at[b], gsem.at[b]
        )

    def write(c):
        b = c % _NBUF
        return pltpu.make_async_copy(
            rows_v.at[b], out_hbm.at[pl.ds(base + c * _CHUNK, _CHUNK)], osem.at[b]
        )

    for c in range(_LOOKAHEAD):
        gather(c).start()
    for c in range(_N_CHUNKS):
        gather(c).wait()
        write(c).start()
        if c + _LOOKAHEAD < _N_CHUNKS:
            if c >= 1:
                write(c - 1).wait()
            gather(c + _LOOKAHEAD).start()
    for c in range(max(0, _N_CHUNKS - _LOOKAHEAD - 1), _N_CHUNKS):
        write(c).wait()


def kernel(x, word_mat):
    idx = x.reshape(_NW, _N_CHUNKS, _CHUNK)
    out = _gather_rows(idx, word_mat)
    return out.reshape(x.shape[0], x.shape[1], _D)
